# Initial kernel scaffold; baseline (speedup 1.0000x reference)
import jax
import jax.numpy as jnp
from jax.experimental import pallas as pl


def kernel(x, edge_index, edge_attr, A, W_gcn, b_gcn, W1, b1, W2, b2):
    raise NotImplementedError("write your pallas kernel here")


# speedup vs baseline: 8.6515x; 8.6515x over previous
import functools

import jax
import jax.numpy as jnp
from jax import lax
from jax.experimental import pallas as pl
from jax.experimental.pallas import tpu as pltpu
from jax.experimental.pallas import tpu_sc as plsc

N = 10000
NP = 10240
E = 320000
E2 = E + NP
D = 128
HD = 128
MD = 128
KD = 64

NC = 2
NS = 16
NWRK = NC * NS
EPW = E2 // NWRK
WIN = 80
NWIN = EPW // WIN
RPT = NP // NS

@functools.cache
def _sc_mesh():
    return plsc.VectorSubcoreMesh(core_axis_name="c", subcore_axis_name="s",
                                  num_cores=NC, num_subcores=NS)


def _deg_body(dst_hbm, ew_hbm, out_hbm, idx_v, ew_v, zeros_v, acc_sh):
    cid = lax.axis_index("c")
    sid = lax.axis_index("s")
    wid = sid * NC + cid

    def _z(i, _):
        zeros_v[pl.ds(i * 16, 16)] = jnp.zeros((16,), jnp.float32)
        return _
    lax.fori_loop(0, RPT // 16, _z, 0)
    pltpu.sync_copy(zeros_v, acc_sh.at[pl.ds(sid * RPT, RPT)])
    plsc.subcore_barrier()

    base0 = wid * EPW

    def _body(w, _):
        base = pl.multiple_of(base0 + w * WIN, 8)
        pltpu.sync_copy(dst_hbm.at[pl.ds(base, WIN)], idx_v)
        pltpu.sync_copy(ew_hbm.at[pl.ds(base, WIN)], ew_v)
        pltpu.sync_copy(ew_v, acc_sh.at[idx_v], add=True)
        return _
    lax.fori_loop(0, NWIN, _body, 0)
    plsc.subcore_barrier()

    pltpu.sync_copy(acc_sh.at[pl.ds(sid * RPT, RPT)],
                    out_hbm.at[cid, pl.ds(sid * RPT, RPT)])


def _msg_body(src_hbm, dst_hbm, ew_hbm, xw_hbm, dis_hbm, out_hbm,
              dis_v, sidx_v, didx_v, ew_v, norm_v, rows_v, zeros_v, acc_sh,
              sem):
    cid = lax.axis_index("c")
    sid = lax.axis_index("s")
    wid = sid * NC + cid

    def _z1(i, _):
        r = i // (D // 16)
        c = i % (D // 16)
        zeros_v[r, pl.ds(c * 16, 16)] = jnp.zeros((16,), jnp.float32)
        return _
    lax.fori_loop(0, WIN * (D // 16), _z1, 0)
    for j in range(RPT // WIN):
        pltpu.sync_copy(zeros_v, acc_sh.at[pl.ds(sid * RPT + j * WIN, WIN)])

    pltpu.sync_copy(dis_hbm, dis_v)
    plsc.subcore_barrier()

    base0 = wid * EPW

    def _body(w, _):
        base = pl.multiple_of(base0 + w * WIN, 8)
        pltpu.sync_copy(src_hbm.at[pl.ds(base, WIN)], sidx_v)
        pltpu.sync_copy(dst_hbm.at[pl.ds(base, WIN)], didx_v)
        pltpu.sync_copy(ew_hbm.at[pl.ds(base, WIN)], ew_v)
        pltpu.async_copy(xw_hbm.at[sidx_v], rows_v, sem).wait()
        for j in range(WIN // 16):
            s16 = sidx_v[pl.ds(j * 16, 16)]
            d16 = didx_v[pl.ds(j * 16, 16)]
            nrm = (plsc.load_gather(dis_v, [s16])
                   * plsc.load_gather(dis_v, [d16])
                   * ew_v[pl.ds(j * 16, 16)])
            norm_v[pl.ds(16 + j * 16, 16)] = nrm
        for r in range(WIN):
            nb = plsc.load_gather(norm_v, [jnp.full((16,), 16 + r, jnp.int32)])
            for c in range(D // 16):
                rows_v[r, pl.ds(c * 16, 16)] = rows_v[r, pl.ds(c * 16, 16)] * nb
        pltpu.sync_copy(rows_v, acc_sh.at[didx_v], add=True)
        return _
    lax.fori_loop(0, NWIN, _body, 0)
    plsc.subcore_barrier()

    for j in range(RPT // WIN):
        pltpu.sync_copy(acc_sh.at[pl.ds(sid * RPT + j * WIN, WIN)],
                        out_hbm.at[cid, pl.ds(sid * RPT + j * WIN, WIN)])


@functools.cache
def _deg_kernel():
    return pl.kernel(
        _deg_body,
        out_type=jax.ShapeDtypeStruct((NC, NP), jnp.float32),
        mesh=_sc_mesh(),
        compiler_params=pltpu.CompilerParams(needs_layout_passes=False),
        scratch_types=[
            pltpu.VMEM((WIN,), jnp.int32),
            pltpu.VMEM((WIN,), jnp.float32),
            pltpu.VMEM((RPT,), jnp.float32),
            pltpu.VMEM_SHARED((NP,), jnp.float32),
        ],
    )


@functools.cache
def _msg_kernel():
    return pl.kernel(
        _msg_body,
        out_type=jax.ShapeDtypeStruct((NC, NP, D), jnp.float32),
        mesh=_sc_mesh(),
        compiler_params=pltpu.CompilerParams(needs_layout_passes=False),
        scratch_types=[
            pltpu.VMEM((NP,), jnp.float32),
            pltpu.VMEM((WIN,), jnp.int32),
            pltpu.VMEM((WIN,), jnp.int32),
            pltpu.VMEM((WIN,), jnp.float32),
            pltpu.VMEM((WIN + 16,), jnp.float32),
            pltpu.VMEM((WIN, D), jnp.float32),
            pltpu.VMEM((WIN, D), jnp.float32),
            pltpu.VMEM_SHARED((NP, D), jnp.float32),
            pltpu.SemaphoreType.DMA,
        ],
    )


def _tc_a_body(x_ref, w_ref, deg_ref, xw_ref, dis_ref):
    xw_ref[...] = jnp.dot(x_ref[...], w_ref[...],
                          preferred_element_type=jnp.float32,
                          precision=lax.Precision.HIGHEST)
    dsum = deg_ref[0] + deg_ref[1]
    dis_ref[...] = lax.rsqrt(dsum)


_BLK = 1024
_GRID = NP // _BLK


def _tc_a(x_pad, w_gcn, deg):
    return pl.pallas_call(
        _tc_a_body,
        grid=(_GRID,),
        in_specs=[
            pl.BlockSpec((_BLK, D), lambda i: (i, 0)),
            pl.BlockSpec((D, HD), lambda i: (0, 0)),
            pl.BlockSpec((NC, _BLK // 128, 128), lambda i: (0, i, 0)),
        ],
        out_specs=[
            pl.BlockSpec((_BLK, HD), lambda i: (i, 0)),
            pl.BlockSpec((_BLK // 128, 128), lambda i: (i, 0)),
        ],
        out_shape=[
            jax.ShapeDtypeStruct((NP, HD), jnp.float32),
            jax.ShapeDtypeStruct((NP // 128, 128), jnp.float32),
        ],
    )(x_pad, w_gcn, deg)


def _elu(v):
    return jnp.where(v > 0, v, jnp.exp(jnp.minimum(v, 0.0)) - 1.0)


def _tc_b_body(y_ref, bg_ref, w1_ref, b1_ref, w2_ref, b2_ref, s_ref):
    h = y_ref[0] + y_ref[1] + bg_ref[...]
    h = _elu(h)
    h1 = _elu(jnp.dot(h, w1_ref[...], preferred_element_type=jnp.float32,
                      precision=lax.Precision.HIGHEST) + b1_ref[...])
    hl = jnp.dot(h1, w2_ref[...], preferred_element_type=jnp.float32,
                 precision=lax.Precision.HIGHEST) + b2_ref[...]
    m = jnp.max(hl, axis=-1, keepdims=True)
    e = jnp.exp(hl - m)
    s_ref[...] = e / jnp.sum(e, axis=-1, keepdims=True)


def _tc_b(y, b_gcn, w1, b1, w2, b2):
    return pl.pallas_call(
        _tc_b_body,
        grid=(_GRID,),
        in_specs=[
            pl.BlockSpec((NC, _BLK, HD), lambda i: (0, i, 0)),
            pl.BlockSpec((1, HD), lambda i: (0, 0)),
            pl.BlockSpec((HD, MD), lambda i: (0, 0)),
            pl.BlockSpec((1, MD), lambda i: (0, 0)),
            pl.BlockSpec((MD, KD), lambda i: (0, 0)),
            pl.BlockSpec((1, KD), lambda i: (0, 0)),
        ],
        out_specs=pl.BlockSpec((_BLK, KD), lambda i: (i, 0)),
        out_shape=jax.ShapeDtypeStruct((NP, KD), jnp.float32),
    )(y, b_gcn, w1, b1, w2, b2)


def kernel(x, edge_index, edge_attr, A, W_gcn, b_gcn, W1, b1, W2, b2):
    loop = jnp.arange(NP, dtype=jnp.int32)
    src2 = jnp.concatenate([edge_index[0], loop])
    dst2 = jnp.concatenate([edge_index[1], loop])
    ew2 = jnp.concatenate([edge_attr, jnp.ones((NP,), jnp.float32)])
    x_pad = jnp.concatenate([x, jnp.zeros((NP - N, D), jnp.float32)], axis=0)

    deg = _deg_kernel()(dst2, ew2)
    xw, dis = _tc_a(x_pad, W_gcn, deg.reshape(NC, NP // 128, 128))
    y = _msg_kernel()(src2, dst2, ew2, xw, dis.reshape(NP))
    s_pad = _tc_b(y, b_gcn.reshape(1, HD), W1, b1.reshape(1, MD),
                  W2, b2.reshape(1, KD))
    return (A, s_pad[:N])


# speedup vs baseline: 9.5936x; 1.1089x over previous
import functools

import jax
import jax.numpy as jnp
from jax import lax
from jax.experimental import pallas as pl
from jax.experimental.pallas import tpu as pltpu
from jax.experimental.pallas import tpu_sc as plsc

N = 10000
NP = 10240
E = 320000
E2 = E + NP
D = 128
HD = 128
MD = 128
KD = 64

NC = 2
NS = 16
NWRK = NC * NS
EPW = E2 // NWRK
WIN = 80
NWIN = EPW // WIN
RPT = NP // NS

@functools.cache
def _sc_mesh():
    return plsc.VectorSubcoreMesh(core_axis_name="c", subcore_axis_name="s",
                                  num_cores=NC, num_subcores=NS)


def _deg_body(eidx_hbm, ewp_hbm, out_hbm, eidx_v, ewp_v, zeros_v, acc_sh,
              dsem):
    cid = lax.axis_index("c")
    sid = lax.axis_index("s")
    wid = sid * NC + cid

    def _z(i, _):
        zeros_v[pl.ds(i * 16, 16)] = jnp.zeros((16,), jnp.float32)
        return _
    lax.fori_loop(0, RPT // 16, _z, 0)
    pltpu.sync_copy(zeros_v, acc_sh.at[pl.ds(sid * RPT, RPT)])

    pltpu.sync_copy(eidx_hbm.at[wid], eidx_v)
    pltpu.sync_copy(ewp_hbm.at[wid], ewp_v)
    plsc.subcore_barrier()

    for b in range(8):
        for k in range(16):
            w = 16 * b + k
            pltpu.async_copy(ewp_v.at[w], acc_sh.at[eidx_v.at[w, 1]], dsem,
                             add=True)
        for k in range(16):
            w = 16 * b + k
            pltpu.make_async_copy(ewp_v.at[w], acc_sh.at[eidx_v.at[w, 1]],
                                  dsem).wait()
    pltpu.async_copy(ewp_v.at[NWIN - 1], acc_sh.at[eidx_v.at[NWIN - 1, 1]],
                     dsem, add=True).wait()
    plsc.subcore_barrier()

    pltpu.sync_copy(acc_sh.at[pl.ds(sid * RPT, RPT)],
                    out_hbm.at[cid, pl.ds(sid * RPT, RPT)])


def _msg_body(src_hbm, dst_hbm, ew_hbm, xw_hbm, dis_hbm, out_hbm,
              dis_v, sidx_v, didx_v, ew_v, norm_v, rows_v, zeros_v, acc_sh,
              sem):
    cid = lax.axis_index("c")
    sid = lax.axis_index("s")
    wid = sid * NC + cid

    def _z1(i, _):
        r = i // (D // 16)
        c = i % (D // 16)
        zeros_v[r, pl.ds(c * 16, 16)] = jnp.zeros((16,), jnp.float32)
        return _
    lax.fori_loop(0, WIN * (D // 16), _z1, 0)
    for j in range(RPT // WIN):
        pltpu.sync_copy(zeros_v, acc_sh.at[pl.ds(sid * RPT + j * WIN, WIN)])

    pltpu.sync_copy(dis_hbm, dis_v)
    plsc.subcore_barrier()

    base0 = wid * EPW

    def _body(w, _):
        base = pl.multiple_of(base0 + w * WIN, 8)
        pltpu.sync_copy(src_hbm.at[pl.ds(base, WIN)], sidx_v)
        pltpu.sync_copy(dst_hbm.at[pl.ds(base, WIN)], didx_v)
        pltpu.sync_copy(ew_hbm.at[pl.ds(base, WIN)], ew_v)
        pltpu.async_copy(xw_hbm.at[sidx_v], rows_v, sem).wait()
        for j in range(WIN // 16):
            s16 = sidx_v[pl.ds(j * 16, 16)]
            d16 = didx_v[pl.ds(j * 16, 16)]
            nrm = (plsc.load_gather(dis_v, [s16])
                   * plsc.load_gather(dis_v, [d16])
                   * ew_v[pl.ds(j * 16, 16)])
            norm_v[pl.ds(16 + j * 16, 16)] = nrm
        for r in range(WIN):
            nb = plsc.load_gather(norm_v, [jnp.full((16,), 16 + r, jnp.int32)])
            for c in range(D // 16):
                rows_v[r, pl.ds(c * 16, 16)] = rows_v[r, pl.ds(c * 16, 16)] * nb
        pltpu.sync_copy(rows_v, acc_sh.at[didx_v], add=True)
        return _
    lax.fori_loop(0, NWIN, _body, 0)
    plsc.subcore_barrier()

    for j in range(RPT // WIN):
        pltpu.sync_copy(acc_sh.at[pl.ds(sid * RPT + j * WIN, WIN)],
                        out_hbm.at[cid, pl.ds(sid * RPT + j * WIN, WIN)])


@functools.cache
def _deg_kernel():
    return pl.kernel(
        _deg_body,
        out_type=jax.ShapeDtypeStruct((NC, NP), jnp.float32),
        mesh=_sc_mesh(),
        compiler_params=pltpu.CompilerParams(needs_layout_passes=False),
        scratch_types=[
            pltpu.VMEM((NWIN, 2, WIN), jnp.int32),
            pltpu.VMEM((NWIN, WIN), jnp.float32),
            pltpu.VMEM((RPT,), jnp.float32),
            pltpu.VMEM_SHARED((NP,), jnp.float32),
            pltpu.SemaphoreType.DMA,
        ],
    )


@functools.cache
def _msg_kernel():
    return pl.kernel(
        _msg_body,
        out_type=jax.ShapeDtypeStruct((NC, NP, D), jnp.float32),
        mesh=_sc_mesh(),
        compiler_params=pltpu.CompilerParams(needs_layout_passes=False),
        scratch_types=[
            pltpu.VMEM((NP,), jnp.float32),
            pltpu.VMEM((WIN,), jnp.int32),
            pltpu.VMEM((WIN,), jnp.int32),
            pltpu.VMEM((WIN,), jnp.float32),
            pltpu.VMEM((WIN + 16,), jnp.float32),
            pltpu.VMEM((WIN, D), jnp.float32),
            pltpu.VMEM((WIN, D), jnp.float32),
            pltpu.VMEM_SHARED((NP, D), jnp.float32),
            pltpu.SemaphoreType.DMA,
        ],
    )


def _tc_a_body(x_ref, w_ref, deg_ref, xw_ref, dis_ref):
    xw_ref[...] = jnp.dot(x_ref[...], w_ref[...],
                          preferred_element_type=jnp.float32,
                          precision=lax.Precision.HIGHEST)
    dsum = deg_ref[0] + deg_ref[1]
    dis_ref[...] = lax.rsqrt(dsum)


_BLK = 1024
_GRID = NP // _BLK


def _tc_a(x_pad, w_gcn, deg):
    return pl.pallas_call(
        _tc_a_body,
        grid=(_GRID,),
        in_specs=[
            pl.BlockSpec((_BLK, D), lambda i: (i, 0)),
            pl.BlockSpec((D, HD), lambda i: (0, 0)),
            pl.BlockSpec((NC, _BLK // 128, 128), lambda i: (0, i, 0)),
        ],
        out_specs=[
            pl.BlockSpec((_BLK, HD), lambda i: (i, 0)),
            pl.BlockSpec((_BLK // 128, 128), lambda i: (i, 0)),
        ],
        out_shape=[
            jax.ShapeDtypeStruct((NP, HD), jnp.float32),
            jax.ShapeDtypeStruct((NP // 128, 128), jnp.float32),
        ],
    )(x_pad, w_gcn, deg)


def _elu(v):
    return jnp.where(v > 0, v, jnp.exp(jnp.minimum(v, 0.0)) - 1.0)


def _tc_b_body(y_ref, bg_ref, w1_ref, b1_ref, w2_ref, b2_ref, s_ref):
    h = y_ref[0] + y_ref[1] + bg_ref[...]
    h = _elu(h)
    h1 = _elu(jnp.dot(h, w1_ref[...], preferred_element_type=jnp.float32,
                      precision=lax.Precision.HIGHEST) + b1_ref[...])
    hl = jnp.dot(h1, w2_ref[...], preferred_element_type=jnp.float32,
                 precision=lax.Precision.HIGHEST) + b2_ref[...]
    m = jnp.max(hl, axis=-1, keepdims=True)
    e = jnp.exp(hl - m)
    s_ref[...] = e / jnp.sum(e, axis=-1, keepdims=True)


def _tc_b(y, b_gcn, w1, b1, w2, b2):
    return pl.pallas_call(
        _tc_b_body,
        grid=(_GRID,),
        in_specs=[
            pl.BlockSpec((NC, _BLK, HD), lambda i: (0, i, 0)),
            pl.BlockSpec((1, HD), lambda i: (0, 0)),
            pl.BlockSpec((HD, MD), lambda i: (0, 0)),
            pl.BlockSpec((1, MD), lambda i: (0, 0)),
            pl.BlockSpec((MD, KD), lambda i: (0, 0)),
            pl.BlockSpec((1, KD), lambda i: (0, 0)),
        ],
        out_specs=pl.BlockSpec((_BLK, KD), lambda i: (i, 0)),
        out_shape=jax.ShapeDtypeStruct((NP, KD), jnp.float32),
    )(y, b_gcn, w1, b1, w2, b2)


def kernel(x, edge_index, edge_attr, A, W_gcn, b_gcn, W1, b1, W2, b2):
    loop = jnp.arange(NP, dtype=jnp.int32)
    src2 = jnp.concatenate([edge_index[0], loop])
    dst2 = jnp.concatenate([edge_index[1], loop])
    ew2 = jnp.concatenate([edge_attr, jnp.ones((NP,), jnp.float32)])
    x_pad = jnp.concatenate([x, jnp.zeros((NP - N, D), jnp.float32)], axis=0)

    eidx = jnp.stack([src2.reshape(NWRK, NWIN, WIN),
                      dst2.reshape(NWRK, NWIN, WIN)], axis=2)
    ewp = ew2.reshape(NWRK, NWIN, WIN)
    deg = _deg_kernel()(eidx, ewp)
    xw, dis = _tc_a(x_pad, W_gcn, deg.reshape(NC, NP // 128, 128))
    y = _msg_kernel()(src2, dst2, ew2, xw, dis.reshape(NP))
    s_pad = _tc_b(y, b_gcn.reshape(1, HD), W1, b1.reshape(1, MD),
                  W2, b2.reshape(1, KD))
    return (A, s_pad[:N])
